# Initial kernel scaffold; baseline (speedup 1.0000x reference)
import jax
import jax.numpy as jnp
from jax.experimental import pallas as pl


def kernel(input, clstoken, position_table):
    raise NotImplementedError("write your pallas kernel here")


# speedup vs baseline: 3.7091x; 3.7091x over previous
import functools

import jax
import jax.numpy as jnp
from jax import lax
from jax.experimental import pallas as pl
from jax.experimental.pallas import tpu as pltpu
from jax.experimental.pallas import tpu_sc as plsc

B, S, D = 1024, 200, 128
T = S + 1
IN_W = S * D
OUT_W = T * D
NC, NS = 2, 16
NW = NC * NS
NB = B // NW
NBUF = 4
LANES = 16


def _sc_body(in_hbm, cls_hbm, pos_hbm, out_hbm,
             pos_v, cls_v, buf0, buf1, buf2, buf3,
             gs0, gs1, gs2, gs3, ss0, ss1, ss2, ss3):
    bufs = (buf0, buf1, buf2, buf3)
    gsems = (gs0, gs1, gs2, gs3)
    ssems = (ss0, ss1, ss2, ss3)

    wid = lax.axis_index("s") * NC + lax.axis_index("c")
    base = wid * NB

    pltpu.sync_copy(pos_hbm, pos_v)
    pltpu.sync_copy(cls_hbm, cls_v)
    for i in range(D // LANES):
        sl = pl.ds(i * LANES, LANES)
        pos_v[sl] = pos_v[sl] + cls_v[sl]
    for j in range(NBUF):
        for i in range(D // LANES):
            sl = pl.ds(i * LANES, LANES)
            bufs[j][sl] = pos_v[sl]

    def start_gather(b, j):
        pltpu.async_copy(in_hbm.at[pl.ds(b * IN_W, IN_W)],
                         bufs[j].at[pl.ds(D, IN_W)], gsems[j])

    def wait_gather(j):
        pltpu.make_async_copy(in_hbm.at[pl.ds(0, IN_W)],
                              bufs[j].at[pl.ds(D, IN_W)], gsems[j]).wait()

    def start_scatter(b, j):
        pltpu.async_copy(bufs[j], out_hbm.at[pl.ds(b * OUT_W, OUT_W)],
                         ssems[j])

    def wait_scatter(j):
        pltpu.make_async_copy(bufs[j], out_hbm.at[pl.ds(0, OUT_W)],
                              ssems[j]).wait()

    start_gather(base, 0)
    start_gather(base + 1, 1)

    @pl.loop(0, NB, step=NBUF)
    def _(g):
        for j in range(NBUF):
            b = g + j
            wait_gather(j)

            @plsc.parallel_loop(D, OUT_W, LANES, unroll=8)
            def _(off):
                plsc.addupdate(bufs[j].at[pl.ds(off, LANES)],
                               pos_v[pl.ds(off, LANES)])

            start_scatter(base + b, j)

            nxt = (j + 2) % NBUF

            @pl.when(b + 2 < NB)
            def _():
                @pl.when(b >= 2)
                def _():
                    wait_scatter(nxt)
                start_gather(base + b + 2, nxt)

    for j in range(NBUF):
        wait_scatter(j)


@jax.jit
def _run(inp, cls, pos):
    mesh = plsc.VectorSubcoreMesh(core_axis_name="c", subcore_axis_name="s")
    return pl.kernel(
        _sc_body,
        out_type=jax.ShapeDtypeStruct((B * T * D,), jnp.float32),
        mesh=mesh,
        scratch_types=[
            pltpu.VMEM((OUT_W,), jnp.float32),
            pltpu.VMEM((D,), jnp.float32),
            pltpu.VMEM((OUT_W,), jnp.float32),
            pltpu.VMEM((OUT_W,), jnp.float32),
            pltpu.VMEM((OUT_W,), jnp.float32),
            pltpu.VMEM((OUT_W,), jnp.float32),
            pltpu.SemaphoreType.DMA,
            pltpu.SemaphoreType.DMA,
            pltpu.SemaphoreType.DMA,
            pltpu.SemaphoreType.DMA,
            pltpu.SemaphoreType.DMA,
            pltpu.SemaphoreType.DMA,
            pltpu.SemaphoreType.DMA,
            pltpu.SemaphoreType.DMA,
        ],
    )(inp, cls, pos)


def kernel(input, clstoken, position_table):
    out = _run(input.reshape(-1), clstoken.reshape(-1),
               position_table.reshape(-1))
    return out.reshape(B, T, D)
